# Initial kernel scaffold; baseline (speedup 1.0000x reference)
import jax
import jax.numpy as jnp
from jax.experimental import pallas as pl


def kernel(h_paper, h_author, edge_index_writes, edge_index_cites, Wk, bk, Wq, bq, Wv, bv, Wa, ba, rel_pri, rel_att, rel_msg, skip):
    raise NotImplementedError("write your pallas kernel here")


# speedup vs baseline: 36.5263x; 36.5263x over previous
import functools
import math

import numpy as np

import jax
import jax.numpy as jnp
from jax import lax
from jax.experimental import pallas as pl
from jax.experimental.pallas import tpu as pltpu
from jax.experimental.pallas import tpu_sc as plsc

H = 8
DK = 16
D = 128
SQRT_DK = math.sqrt(DK)
NC = 2
NS = 16
NW = NC * NS
CHUNK = 40

_f32 = jnp.float32

_PERM = np.zeros((D,), np.int32)
for _c in range(8):
    for _l in range(16):
        _h = _l if _l < 8 else 15 - _l
        _d = 2 * _c + (1 if _l >= 8 else 0)
        _PERM[_c * 16 + _l] = _h * 16 + _d
_PERM_A = tuple(int(x) for x in _PERM)


def _fold_body(wk_ref, bk_ref, wv_ref, bv_ref, att_ref, msg_ref, pri_ref,
               kw_w, kw_b, vw_w, vw_b, kc_w, kc_b, vc_w, vc_b):
    pri = pri_ref[...]
    att = att_ref[...]
    msg = msg_ref[...]
    wk = wk_ref[...]
    bk = bk_ref[...]
    wv = wv_ref[...]
    bv = bv_ref[...]

    def fold(w, b, rel, out_w, out_b, scales):
        cols_w, cols_b = [], []
        for h in range(H):
            rh = rel[h]
            if scales is not None:
                rh = rh * scales[h]
            cols_w.append(jnp.dot(w[:, h * DK:(h + 1) * DK], rh,
                                  preferred_element_type=_f32))
            cols_b.append(jnp.dot(b[:, h * DK:(h + 1) * DK], rh,
                                  preferred_element_type=_f32))
        out_w[...] = jnp.concatenate(cols_w, axis=1)
        out_b[...] = jnp.concatenate(cols_b, axis=1)

    sc0 = [pri[0, h] / SQRT_DK for h in range(H)]
    sc1 = [pri[1, h] / SQRT_DK for h in range(H)]
    fold(wk[1], bk[1:2], att[0], kw_w, kw_b, sc0)
    fold(wv[1], bv[1:2], msg[0], vw_w, vw_b, None)
    fold(wk[0], bk[0:1], att[1], kc_w, kc_b, sc1)
    fold(wv[0], bv[0:1], msg[1], vc_w, vc_b, None)


def _fold_weights(Wk, bk, Wv, bv, rel_att, rel_msg, rel_pri):
    shp_w = jax.ShapeDtypeStruct((D, D), _f32)
    shp_b = jax.ShapeDtypeStruct((1, D), _f32)
    return pl.pallas_call(
        _fold_body,
        out_shape=[shp_w, shp_b, shp_w, shp_b, shp_w, shp_b, shp_w, shp_b],
    )(Wk, bk, Wv, bv, rel_att, rel_msg, rel_pri)


def _proj_body(hp_ref, ha_ref, wq_ref, bq_ref,
               kww_ref, kwb_ref, vww_ref, vwb_ref,
               kcw_ref, kcb_ref, vcw_ref, vcb_ref,
               q_out, kw_out, vw_out, kc_out, vc_out,
               mq2_out, mkw2_out, mkc2_out):
    i = pl.program_id(0)
    hp = hp_ref[...]
    ha = ha_ref[...]
    q = jnp.dot(hp, wq_ref[...], preferred_element_type=_f32) + bq_ref[...]
    kw = jnp.dot(ha, kww_ref[...], preferred_element_type=_f32) + kwb_ref[...]
    vw = jnp.dot(ha, vww_ref[...], preferred_element_type=_f32) + vwb_ref[...]
    kc = jnp.dot(hp, kcw_ref[...], preferred_element_type=_f32) + kcb_ref[...]
    vc = jnp.dot(hp, vcw_ref[...], preferred_element_type=_f32) + vcb_ref[...]
    q_out[...] = q
    kw_out[...] = kw
    vw_out[...] = vw
    kc_out[...] = kc
    vc_out[...] = vc

    def head_norm2(x):
        sq = x * x
        acc = sq[:, 0:16]
        for c in range(1, 8):
            acc = acc + sq[:, c * 16:(c + 1) * 16]
        return jnp.max(acc, axis=0, keepdims=True)

    bq2 = head_norm2(q)
    bkw2 = head_norm2(kw)
    bkc2 = head_norm2(kc)

    @pl.when(i == 0)
    def _():
        mq2_out[...] = bq2
        mkw2_out[...] = bkw2
        mkc2_out[...] = bkc2

    @pl.when(i > 0)
    def _():
        mq2_out[...] = jnp.maximum(mq2_out[...], bq2)
        mkw2_out[...] = jnp.maximum(mkw2_out[...], bkw2)
        mkc2_out[...] = jnp.maximum(mkc2_out[...], bkc2)


def _project(hp, ha, wq, bq, kww, kwb, vww, vwb, kcw, kcb, vcw, vcb, blk):
    n = hp.shape[0]
    grid = (n // blk,)
    row = pl.BlockSpec((blk, D), lambda i: (i, 0))
    full = lambda a: pl.BlockSpec(a.shape, lambda i: (0,) * a.ndim)
    scal = pl.BlockSpec((1, 16), lambda i: (0, 0))
    return pl.pallas_call(
        _proj_body,
        grid=grid,
        in_specs=[row, row, full(wq), full(bq), full(kww), full(kwb),
                  full(vww), full(vwb), full(kcw), full(kcb),
                  full(vcw), full(vcb)],
        out_specs=[row, row, row, row, row, scal, scal, scal],
        out_shape=[jax.ShapeDtypeStruct((n, D), _f32)] * 5
        + [jax.ShapeDtypeStruct((1, 16), _f32)] * 3,
    )(hp, ha, wq, bq, kww, kwb, vww, vwb, kcw, kcb, vcw, vcb)


def _bound_body(mq2_ref, mkw2_ref, mkc2_ref, mw_out, mc_out):
    q2 = mq2_ref[...]
    mw_out[...] = jnp.sqrt(q2 * mkw2_ref[...])
    mc_out[...] = jnp.sqrt(q2 * mkc2_ref[...])


def _bounds(mq2, mkw2, mkc2):
    return pl.pallas_call(
        _bound_body,
        out_shape=[jax.ShapeDtypeStruct((1, 16), _f32)] * 2,
    )(mq2, mkw2, mkc2)


def _make_edge_kernel(E, N):
    EPW = E // NW
    ITERS = EPW // CHUNK
    ND = -(-N // (8 * NW)) * NW
    FL_T = 10
    FL_R = N // FL_T
    DN_R = ND // FL_T
    ZR = 8

    mesh = plsc.VectorSubcoreMesh(core_axis_name="c", subcore_axis_name="s",
                                  num_cores=NC, num_subcores=NS)

    @functools.partial(
        pl.kernel,
        out_type=[
            jax.ShapeDtypeStruct((NC, 2 * N, D), _f32),
            jax.ShapeDtypeStruct((NC, 2 * ND, D), _f32),
        ],
        mesh=mesh,
        scratch_types=[
            pltpu.VMEM((CHUNK,), jnp.int32),
            pltpu.VMEM((CHUNK,), jnp.int32),
            pltpu.VMEM((CHUNK,), jnp.int32),
            pltpu.VMEM((CHUNK,), jnp.int32),
            pltpu.VMEM((CHUNK, D), _f32),
            pltpu.VMEM((CHUNK, D), _f32),
            pltpu.VMEM((CHUNK, D), _f32),
            pltpu.VMEM((CHUNK, D), _f32),
            pltpu.VMEM((1, D), _f32),
            pltpu.VMEM((ZR, D), _f32),
            pltpu.VMEM_SHARED((N, D), _f32),
            pltpu.VMEM_SHARED((ND, D), _f32),
            pltpu.SemaphoreType.DMA,
            pltpu.SemaphoreType.DMA,
            pltpu.SemaphoreType.DMA,
        ],
    )
    def edge_kernel(tab_hbm, idx_hbm,
                    agg_o, den_o,
                    idx_s, idx_d, idx_v, idx_n, krows, qrows, vrows, exbuf,
                    mbuf, zbuf, sp_agg, sp_den, sem1, sem2, sem3):
        c = lax.axis_index("c")
        s = lax.axis_index("s")
        wid = s * NC + c
        zerov = jnp.zeros((16,), _f32)

        def zb(i, _):
            for jc in range(D // 16):
                zbuf[i, pl.ds(jc * 16, 16)] = zerov
            return 0
        lax.fori_loop(0, ZR, zb, 0)


        def one_rel(r, agg_base, den_base):
            @pl.when(s < FL_T)
            def _():
                for j in range(FL_R // ZR):
                    pltpu.sync_copy(zbuf, sp_agg.at[pl.ds(s * FL_R + j * ZR, ZR)])
                for j in range(DN_R // ZR):
                    pltpu.sync_copy(zbuf, sp_den.at[pl.ds(s * DN_R + j * ZR, ZR)])
            pltpu.sync_copy(tab_hbm.at[pl.ds(5 * N + r, 1)], mbuf)
            mvec = mbuf[0, pl.ds(0, 16)]
            plsc.subcore_barrier()

            ebase = wid * EPW

            def chunk_body(it, _):
                base = 4 * r * E + ebase + it * CHUNK
                pltpu.sync_copy(idx_hbm.at[pl.ds(base, CHUNK)], idx_d)
                pltpu.sync_copy(idx_hbm.at[pl.ds(base + E, CHUNK)], idx_s)
                pltpu.sync_copy(idx_hbm.at[pl.ds(base + 2 * E, CHUNK)], idx_v)
                pltpu.sync_copy(idx_hbm.at[pl.ds(base + 3 * E, CHUNK)], idx_n)
                cp1 = pltpu.async_copy(tab_hbm.at[idx_s], krows, sem1)
                cp2 = pltpu.async_copy(tab_hbm.at[idx_d], qrows, sem2)
                cp3 = pltpu.async_copy(tab_hbm.at[idx_v], vrows, sem3)
                cp1.wait()
                cp2.wait()
                cp3.wait()

                def do_edges(off, lane0, count):
                    dvec = idx_d[pl.ds(off, 16)]
                    slot = dvec & 7
                    for e16 in range(count):
                        e = off + lane0 + e16
                        acc = qrows[e, pl.ds(0, 16)] * krows[e, pl.ds(0, 16)]
                        for ch in range(1, 8):
                            acc = acc + (qrows[e, pl.ds(ch * 16, 16)]
                                         * krows[e, pl.ds(ch * 16, 16)])
                        srow = acc + lax.rev(acc, (0,))
                        ex = jnp.exp(srow - mvec)
                        se = slot[lane0 + e16]
                        for ch in range(8):
                            exbuf[e, pl.ds(ch * 16, 16)] = jnp.where(
                                se == ch, ex, zerov)
                            vrows[e, pl.ds(ch * 16, 16)] = (
                                vrows[e, pl.ds(ch * 16, 16)] * ex)

                def group_body(g, _):
                    do_edges(g * 16, 0, 16)
                    return 0
                lax.fori_loop(0, CHUNK // 16, group_body, 0)
                if CHUNK % 16:
                    tail = CHUNK % 16
                    do_edges(CHUNK - 16, 16 - tail, tail)

                pltpu.sync_copy(vrows, sp_agg.at[idx_d], add=True)
                pltpu.sync_copy(exbuf, sp_den.at[idx_n], add=True)
                return 0
            lax.fori_loop(0, ITERS, chunk_body, 0)
            plsc.subcore_barrier()

            @pl.when(s < FL_T)
            def _():
                pltpu.sync_copy(sp_agg.at[pl.ds(s * FL_R, FL_R)],
                                agg_o.at[c, pl.ds(agg_base + s * FL_R, FL_R)])
                pltpu.sync_copy(sp_den.at[pl.ds(s * DN_R, DN_R)],
                                den_o.at[c, pl.ds(den_base + s * DN_R, DN_R)])
            plsc.subcore_barrier()

        one_rel(0, 0, 0)
        one_rel(1, N, ND)

    return edge_kernel, ND


def _final_body(aggw_ref, denw_ref, aggc_ref, denc_ref, hp_ref,
                wa_ref, ba_ref, skip_ref, out_ref):
    aggw = aggw_ref[0] + aggw_ref[1]
    denw = denw_ref[0] + denw_ref[1]
    aggc = aggc_ref[0] + aggc_ref[1]
    denc = denc_ref[0] + denc_ref[1]

    def norm(agg, den16):
        den16 = jnp.where(den16 > 0, den16, 1.0)
        den_exp = jnp.concatenate([den16] * 8, axis=1)
        return agg / den_exp

    t = 0.5 * (norm(aggw, denw) + norm(aggc, denc))
    trans = jnp.dot(t, wa_ref[...], preferred_element_type=_f32) + ba_ref[...]
    alpha = jax.nn.sigmoid(skip_ref[...][0, 0])
    out_ref[...] = trans * alpha + hp_ref[...] * (1.0 - alpha)


def _finalize(aggw, denw, aggc, denc, hp, wa_perm, ba0, skip2, blk):
    n = hp.shape[0]
    grid = (n // blk,)
    row = pl.BlockSpec((blk, D), lambda i: (i, 0))
    agg_s = pl.BlockSpec((NC, blk, D), lambda i: (0, i, 0))
    den_s = pl.BlockSpec((NC, blk, 16), lambda i: (0, i, 0))
    full = lambda a: pl.BlockSpec(a.shape, lambda i: (0,) * a.ndim)
    return pl.pallas_call(
        _final_body,
        grid=grid,
        in_specs=[agg_s, den_s, agg_s, den_s, row, full(wa_perm), full(ba0),
                  full(skip2)],
        out_specs=row,
        out_shape=jax.ShapeDtypeStruct((n, D), _f32),
    )(aggw, denw, aggc, denc, hp, wa_perm, ba0, skip2)


def kernel(h_paper, h_author, edge_index_writes, edge_index_cites,
           Wk, bk, Wq, bq, Wv, bv, Wa, ba, rel_pri, rel_att, rel_msg, skip):
    n = h_paper.shape[0]
    E = edge_index_writes.shape[1]
    perm = jnp.asarray(_PERM)

    kww, kwb, vww, vwb, kcw, kcb, vcw, vcb = _fold_weights(
        Wk, bk, Wv, bv, rel_att, rel_msg, rel_pri)

    def p2(w, b):
        return jnp.take(w, perm, axis=1), jnp.take(b, perm, axis=1)

    kww, kwb = p2(kww, kwb)
    vww, vwb = p2(vww, vwb)
    kcw, kcb = p2(kcw, kcb)
    vcw, vcb = p2(vcw, vcb)
    wq_p = jnp.take(Wq[0], perm, axis=1)
    bq_p = jnp.take(bq[0], perm, axis=0).reshape(1, D)

    qp, kw, vw, kc, vc, mq2, mkw2, mkc2 = _project(
        h_paper, h_author, wq_p, bq_p, kww, kwb, vww, vwb, kcw, kcb,
        vcw, vcb, blk=400)

    m_w, m_c = _bounds(mq2, mkw2, mkc2)

    edge_kernel, ND = _make_edge_kernel(E, n)
    srcw = edge_index_writes[0]
    dstw = edge_index_writes[1]
    srcc = edge_index_cites[0]
    dstc = edge_index_cites[1]
    tab = jnp.concatenate([qp, kw, kc, vw, vc,
                           jnp.tile(m_w, (1, 8)), jnp.tile(m_c, (1, 8))],
                          axis=0)
    idx_flat = jnp.concatenate([
        dstw, srcw + n, srcw + 3 * n, lax.shift_right_logical(dstw, 3),
        dstc, srcc + 2 * n, srcc + 4 * n, lax.shift_right_logical(dstc, 3),
    ])
    agg_all, den_all = edge_kernel(tab, idx_flat)
    aggw, aggc = agg_all[:, :n], agg_all[:, n:]
    denw, denc = den_all[:, :ND], den_all[:, ND:]

    denw16 = denw.reshape(NC, ND * 8, 16)[:, :n]
    denc16 = denc.reshape(NC, ND * 8, 16)[:, :n]

    wa_perm = Wa[0][perm, :]
    new_paper = _finalize(aggw, denw16, aggc, denc16, h_paper, wa_perm,
                          ba[0].reshape(1, D), skip.reshape(1, 2), blk=400)
    return (new_paper, h_author)
